# baseline (speedup 1.0000x reference)
import functools

import jax
import jax.numpy as jnp
from jax import lax
from jax.experimental import pallas as pl
from jax.experimental.pallas import tpu as pltpu
from jax.experimental.pallas import tpu_sc as plsc

D_MODEL_ = 1024
NUM_EXPERTS_ = 8
D_FF_ = 2048
N_TOK_ = 32
LANES_ = 16
NEG_INF_ = float("-inf")


def _logits_kernel(x_ref, rw_ref, out_ref):
    out_ref[...] = jax.lax.dot_general(
        rw_ref[...], x_ref[...], (((1,), (1,)), ((), ())),
        preferred_element_type=jnp.float32)


def _routing_sc_kernel(lt_hbm, c_hbm, lt_v, c_v):
    cid = lax.axis_index("c")
    sid = lax.axis_index("s")

    @pl.when((sid == 0) & (cid == 0))
    def _():
        pltpu.sync_copy(lt_hbm, lt_v)
        for half in range(N_TOK_ // LANES_):
            m1 = jnp.full((LANES_,), NEG_INF_, jnp.float32)
            m2 = jnp.full((LANES_,), NEG_INF_, jnp.float32)
            i1 = jnp.zeros((LANES_,), jnp.int32)
            i2 = jnp.zeros((LANES_,), jnp.int32)
            for e in range(NUM_EXPERTS_):
                v = lt_v[pl.ds(e * N_TOK_ + half * LANES_, LANES_)]
                ev = jnp.full((LANES_,), e, jnp.int32)
                beats1 = v > m1
                beats2 = v > m2
                i2 = jnp.where(beats1, i1, jnp.where(beats2, ev, i2))
                m2 = jnp.where(beats1, m1, jnp.where(beats2, v, m2))
                i1 = jnp.where(beats1, ev, i1)
                m1 = jnp.where(beats1, v, m1)
            p1 = 1.0 / (1.0 + jnp.exp(m2 - m1))
            p2 = 1.0 - p1
            for e in range(NUM_EXPERTS_):
                ev = jnp.full((LANES_,), e, jnp.int32)
                c_v[pl.ds(e * N_TOK_ + half * LANES_, LANES_)] = (
                    jnp.where(i1 == ev, p1, 0.0)
                    + jnp.where(i2 == ev, p2, 0.0))
        pltpu.sync_copy(c_v, c_hbm)


def _routing_sc(logits_t_flat):
    mesh = plsc.VectorSubcoreMesh(core_axis_name="c", subcore_axis_name="s")
    f = functools.partial(
        pl.kernel,
        mesh=mesh,
        out_type=jax.ShapeDtypeStruct((NUM_EXPERTS_ * N_TOK_,), jnp.float32),
        scratch_types=[
            pltpu.VMEM((NUM_EXPERTS_ * N_TOK_,), jnp.float32),
            pltpu.VMEM((NUM_EXPERTS_ * N_TOK_,), jnp.float32),
        ],
    )(_routing_sc_kernel)
    return f(logits_t_flat)


def _moe_kernel(x_ref, w1_ref, w2_ref, c_ref, out_ref):
    e = pl.program_id(0)
    x = x_ref[...]

    row = jax.lax.broadcasted_iota(jnp.int32, (NUM_EXPERTS_, N_TOK_), 0)
    c_e = jnp.sum(jnp.where(row == e, c_ref[...], 0.0), axis=0)

    w1_e = w1_ref[0]
    w2_e = w2_ref[0]
    h = jax.lax.dot_general(x, w1_e, (((1,), (1,)), ((), ())),
                            preferred_element_type=jnp.float32)
    h = 0.5 * h * (1.0 + jax.lax.erf(h * (2.0 ** -0.5)))
    y = jax.lax.dot_general(h, w2_e, (((1,), (1,)), ((), ())),
                            preferred_element_type=jnp.float32)

    contrib = c_e[:, None] * y

    @pl.when(e == 0)
    def _():
        out_ref[...] = contrib

    @pl.when(e > 0)
    def _():
        out_ref[...] += contrib


@jax.jit
def _moe(x_flat, w1, w2, router_w):
    n = x_flat.shape[0]
    logits_t = pl.pallas_call(
        _logits_kernel,
        out_shape=jax.ShapeDtypeStruct((NUM_EXPERTS_, n), jnp.float32),
    )(x_flat, router_w)
    c = _routing_sc(logits_t.reshape(-1)).reshape(NUM_EXPERTS_, n)
    return pl.pallas_call(
        _moe_kernel,
        grid=(NUM_EXPERTS_,),
        in_specs=[
            pl.BlockSpec((n, D_MODEL_), lambda e: (0, 0)),
            pl.BlockSpec((1, D_FF_, D_MODEL_), lambda e: (e, 0, 0)),
            pl.BlockSpec((1, D_MODEL_, D_FF_), lambda e: (e, 0, 0)),
            pl.BlockSpec((NUM_EXPERTS_, n), lambda e: (0, 0)),
        ],
        out_specs=pl.BlockSpec((n, D_MODEL_), lambda e: (0, 0)),
        out_shape=jax.ShapeDtypeStruct((n, D_MODEL_), jnp.float32),
    )(x_flat, w1, w2, c)


def kernel(x, w1, w2, router_w):
    B, T, D = x.shape
    out = _moe(x.reshape(B * T, D), w1, w2, router_w)
    return out.reshape(B, T, D)
